# baseline (speedup 1.0000x reference)
import functools

import jax
import jax.numpy as jnp
from jax import lax
from jax.experimental import pallas as pl
from jax.experimental.pallas import tpu as pltpu
from jax.experimental.pallas import tpu_sc as plsc

B = 16384
F = 26
D = 16
C = 8
V = 1000000
L = 16
NC = 2
NS = 16
NW = NC * NS
BPW = B // NW
CH = 128
NCHUNK = BPW // CH
ROWS = CH * F

OFF_DD = C * F
OFF_DE = OFF_DD + F
OFF_W0 = OFF_DE + C
NCONST = 243

TCOLS = 12800
TROWS = TCOLS * D // 128


def _tr_body(in_ref, out_ref):
    blk = in_ref[...]
    t = jnp.transpose(blk.reshape(D, TROWS, 8), (1, 2, 0))
    out_ref[...] = t.reshape(TROWS, 128)


def _transpose_table(embT):
    return pl.pallas_call(
        _tr_body,
        grid=(pl.cdiv(V, TCOLS),),
        in_specs=[pl.BlockSpec((D, TCOLS), lambda i: (0, i))],
        out_specs=pl.BlockSpec((TROWS, 128), lambda i: (i, 0)),
        out_shape=jax.ShapeDtypeStruct((V * D // 128, 128), jnp.float32),
    )(embT)


def _fm_body(xf_hbm, emb_hbm, bias_hbm, consts_hbm, out_hbm,
             widx_v, rows_v, bias_v, consts_v, out_v, sem, semi):
    cid = lax.axis_index("c")
    sid = lax.axis_index("s")
    wid = sid * NC + cid
    base = wid * BPW
    pltpu.sync_copy(consts_hbm, consts_v)

    icopies = [
        pltpu.async_copy(xf_hbm.at[pl.ds(f * B + base, BPW)],
                         widx_v.at[pl.ds(f * BPW, BPW)], semi)
        for f in range(F)
    ]
    for cp in icopies:
        cp.wait()

    iota = lax.broadcasted_iota(jnp.int32, (L,), 0)
    fi1 = iota * CH
    fi2 = fi1 + 16 * CH
    m0 = jnp.where(iota < F - L, 1.0, 0.0).astype(jnp.float32)
    w0row = consts_v[OFF_W0, :]
    ddrows = [consts_v[OFF_DD + f, :] for f in range(F)]
    derows = [consts_v[OFF_DE + c, :] for c in range(C)]

    def chunk_body(ch, _):
        copies = []
        for f in range(F):
            isl = widx_v.at[pl.ds(f * BPW, BPW)].at[pl.ds(ch * CH, CH)]
            copies.append(pltpu.async_copy(
                emb_hbm.at[isl], rows_v.at[pl.ds(f * CH, CH)], sem))
            copies.append(pltpu.async_copy(
                bias_hbm.at[isl], bias_v.at[pl.ds(f * CH, CH)], sem))
        for cp in copies:
            cp.wait()

        def q_body(q, _):
            def k_body(k, resvec):
                b0 = q * L + k * 4
                accs = []
                for j in range(4):
                    row = rows_v[b0 + j, :]
                    accs.append(row * row * ddrows[0])
                Pss = [[rows_v[b0 + j, :] * consts_v[c, :] for c in range(C)]
                       for j in range(4)]
                for f in range(1, F):
                    rb = f * CH + b0
                    urows = [consts_v[f * C + c, :] for c in range(C)]
                    for j in range(4):
                        row = rows_v[rb + j, :]
                        accs[j] = accs[j] + row * row * ddrows[f]
                        Ps = Pss[j]
                        for c in range(C):
                            Ps[c] = Ps[c] + row * urows[c]
                for j in range(4):
                    for c in range(C):
                        accs[j] = accs[j] + Pss[j][c] * Pss[j][c] * derows[c]
                for j in range(4):
                    b = b0 + j
                    v1 = plsc.load_gather(bias_v, [b + fi1])
                    v2 = plsc.load_gather(bias_v, [jnp.minimum(b + fi2, ROWS - 1)])
                    tot = accs[j] + (v1 + v2 * m0) + w0row
                    r = jnp.sum(tot)
                    resvec = jnp.where(iota == k * 4 + j, r, resvec)
                return resvec

            resvec = lax.fori_loop(0, 4, k_body, jnp.zeros((L,), jnp.float32))
            out_v[pl.ds(ch * CH + q * L, L)] = resvec
            return _

        lax.fori_loop(0, CH // L, q_body, None)
        return _

    lax.fori_loop(0, NCHUNK, chunk_body, None)
    pltpu.sync_copy(out_v, out_hbm.at[pl.ds(base, BPW)])


def kernel(x, emb_table, bias_table, w0, diag_e, U):
    diag_d = -(diag_e[:, None] * U * U).sum(axis=0)
    ones = jnp.ones((1, L), jnp.float32)
    u_rows = U.T.reshape(-1, 1) * ones
    dd_rows = (0.5 * diag_d).reshape(-1, 1) * ones
    de_rows = (0.5 * diag_e).reshape(-1, 1) * ones
    w0_row = jnp.zeros((1, L), jnp.float32).at[0, 0].set(w0[0])
    consts = jnp.concatenate([u_rows, dd_rows, de_rows, w0_row], axis=0)

    table_rm = _transpose_table(emb_table.T)
    xf = x.astype(jnp.int32).T.reshape(-1)
    bias1d = bias_table.T.reshape(-1)

    mesh = plsc.VectorSubcoreMesh(core_axis_name="c", subcore_axis_name="s")
    fm = functools.partial(
        pl.kernel,
        mesh=mesh,
        compiler_params=pltpu.CompilerParams(
            needs_layout_passes=False, use_tc_tiling_on_sc=False),
        out_type=jax.ShapeDtypeStruct((B,), jnp.float32),
        scratch_types=[
            pltpu.VMEM((F * BPW,), jnp.int32),
            pltpu.VMEM((ROWS, D), jnp.float32),
            pltpu.VMEM((ROWS,), jnp.float32),
            pltpu.VMEM((NCONST, L), jnp.float32),
            pltpu.VMEM((BPW,), jnp.float32),
            pltpu.SemaphoreType.DMA,
            pltpu.SemaphoreType.DMA,
        ],
    )(_fm_body)
    return fm(xf, table_rm.reshape(V, D), bias1d, consts)


# speedup vs baseline: 2.0945x; 2.0945x over previous
import functools

import jax
import jax.numpy as jnp
from jax import lax
from jax.experimental import pallas as pl
from jax.experimental.pallas import tpu as pltpu
from jax.experimental.pallas import tpu_sc as plsc

B = 16384
F = 26
D = 16
C = 8
V = 1000000
L = 16
NC = 2
NS = 16
NW = NC * NS
BPW = B // NW
CH = 128
NCHUNK = BPW // CH
ROWS = CH * F

OFF_DD = C * F
OFF_DE = OFF_DD + F
OFF_W0 = OFF_DE + C
NCONST = 243

TCOLS = 12800
TROWS = TCOLS * D // 128


def _tr_body(in_ref, out_ref):
    blk = in_ref[...]
    t3 = blk.T.reshape(TROWS, 8, D)
    out_ref[...] = jnp.concatenate([t3[:, a, :] for a in range(8)], axis=1)


def _transpose_table(embT):
    return pl.pallas_call(
        _tr_body,
        grid=(pl.cdiv(V, TCOLS),),
        in_specs=[pl.BlockSpec((D, TCOLS), lambda i: (0, i))],
        out_specs=pl.BlockSpec((TROWS, 128), lambda i: (i, 0)),
        out_shape=jax.ShapeDtypeStruct((V * D // 128, 128), jnp.float32),
    )(embT)


def _fm_body(xf_hbm, emb_hbm, bias_hbm, consts_hbm, out_hbm,
             widx_v, rows_v, bias_v, consts_v, out_v, sem, semi):
    cid = lax.axis_index("c")
    sid = lax.axis_index("s")
    wid = sid * NC + cid
    base = wid * BPW
    pltpu.sync_copy(consts_hbm, consts_v)

    icopies = [
        pltpu.async_copy(xf_hbm.at[pl.ds(f * B + base, BPW)],
                         widx_v.at[pl.ds(f * BPW, BPW)], semi)
        for f in range(F)
    ]
    for cp in icopies:
        cp.wait()

    iota = lax.broadcasted_iota(jnp.int32, (L,), 0)
    fi1 = iota * CH
    fi2 = fi1 + 16 * CH
    m0 = jnp.where(iota < F - L, 1.0, 0.0).astype(jnp.float32)
    w0row = consts_v[OFF_W0, :]
    ddrows = [consts_v[OFF_DD + f, :] for f in range(F)]
    derows = [consts_v[OFF_DE + c, :] for c in range(C)]

    def chunk_body(ch, _):
        copies = []
        for f in range(F):
            isl = widx_v.at[pl.ds(f * BPW, BPW)].at[pl.ds(ch * CH, CH)]
            copies.append(pltpu.async_copy(
                emb_hbm.at[isl], rows_v.at[pl.ds(f * CH, CH)], sem))
            copies.append(pltpu.async_copy(
                bias_hbm.at[isl], bias_v.at[pl.ds(f * CH, CH)], sem))
        for cp in copies:
            cp.wait()

        def q_body(q, _):
            def k_body(k, resvec):
                b0 = q * L + k * 4
                accs = []
                for j in range(4):
                    row = rows_v[b0 + j, :]
                    accs.append(row * row * ddrows[0])
                Pss = [[rows_v[b0 + j, :] * consts_v[c, :] for c in range(C)]
                       for j in range(4)]
                for f in range(1, F):
                    rb = f * CH + b0
                    urows = [consts_v[f * C + c, :] for c in range(C)]
                    for j in range(4):
                        row = rows_v[rb + j, :]
                        accs[j] = accs[j] + row * row * ddrows[f]
                        Ps = Pss[j]
                        for c in range(C):
                            Ps[c] = Ps[c] + row * urows[c]
                for j in range(4):
                    for c in range(C):
                        accs[j] = accs[j] + Pss[j][c] * Pss[j][c] * derows[c]
                for j in range(4):
                    b = b0 + j
                    v1 = plsc.load_gather(bias_v, [b + fi1])
                    v2 = plsc.load_gather(bias_v, [jnp.minimum(b + fi2, ROWS - 1)])
                    tot = accs[j] + (v1 + v2 * m0) + w0row
                    r = jnp.sum(tot)
                    resvec = jnp.where(iota == k * 4 + j, r, resvec)
                return resvec

            resvec = lax.fori_loop(0, 4, k_body, jnp.zeros((L,), jnp.float32))
            out_v[pl.ds(ch * CH + q * L, L)] = resvec
            return _

        lax.fori_loop(0, CH // L, q_body, None)
        return _

    lax.fori_loop(0, NCHUNK, chunk_body, None)
    pltpu.sync_copy(out_v, out_hbm.at[pl.ds(base, BPW)])


def kernel(x, emb_table, bias_table, w0, diag_e, U):
    diag_d = -(diag_e[:, None] * U * U).sum(axis=0)
    ones = jnp.ones((1, L), jnp.float32)
    u_rows = U.T.reshape(-1, 1) * ones
    dd_rows = (0.5 * diag_d).reshape(-1, 1) * ones
    de_rows = (0.5 * diag_e).reshape(-1, 1) * ones
    w0_row = jnp.zeros((1, L), jnp.float32).at[0, 0].set(w0[0])
    consts = jnp.concatenate([u_rows, dd_rows, de_rows, w0_row], axis=0)

    table_rm = _transpose_table(emb_table.T)
    xf = x.astype(jnp.int32).T.reshape(-1)
    bias1d = bias_table.T.reshape(-1)

    mesh = plsc.VectorSubcoreMesh(core_axis_name="c", subcore_axis_name="s")
    fm = functools.partial(
        pl.kernel,
        mesh=mesh,
        compiler_params=pltpu.CompilerParams(
            needs_layout_passes=False, use_tc_tiling_on_sc=False),
        out_type=jax.ShapeDtypeStruct((B,), jnp.float32),
        scratch_types=[
            pltpu.VMEM((F * BPW,), jnp.int32),
            pltpu.VMEM((ROWS, D), jnp.float32),
            pltpu.VMEM((ROWS,), jnp.float32),
            pltpu.VMEM((NCONST, L), jnp.float32),
            pltpu.VMEM((BPW,), jnp.float32),
            pltpu.SemaphoreType.DMA,
            pltpu.SemaphoreType.DMA,
        ],
    )(_fm_body)
    return fm(xf, table_rm.reshape(V, D), bias1d, consts)


# speedup vs baseline: 3.9509x; 1.8863x over previous
import functools

import jax
import jax.numpy as jnp
from jax import lax
from jax.experimental import pallas as pl
from jax.experimental.pallas import tpu as pltpu
from jax.experimental.pallas import tpu_sc as plsc

B = 16384
F = 26
D = 16
C = 8
V = 1000000
L = 16
NC = 2
NS = 16
NW = NC * NS
BPW = B // NW
CH = 128
NCHUNK = BPW // CH
ROWS = CH * F

OFF_DD = C * F
OFF_DE = OFF_DD + F
OFF_W0 = OFF_DE + C
NCONST = 243

GRP = 16
TCOLS = GRP * 8 * 128
TROWS = GRP * 128
TSTEPS = (V + TCOLS - 1) // TCOLS
VPAD = TSTEPS * TCOLS


def _tr_body(in_ref, out_ref):
    blk = in_ref[...]
    for c in range(GRP):
        tile = jnp.concatenate(
            [blk[:, (c * 8 + k) * 128:(c * 8 + k + 1) * 128]
             for k in range(8)], axis=0)
        out_ref[c * 128:(c + 1) * 128, :] = tile.T


def _transpose_table(embT):
    return pl.pallas_call(
        _tr_body,
        grid=(TSTEPS,),
        in_specs=[pl.BlockSpec((D, TCOLS), lambda i: (0, i))],
        out_specs=pl.BlockSpec((TROWS, 128), lambda i: (i, 0)),
        out_shape=jax.ShapeDtypeStruct((VPAD * D // 128, 128), jnp.float32),
    )(embT)


def _fm_body(xf_hbm, emb_hbm, bias_hbm, consts_hbm, out_hbm,
             widx_v, widx2_v, rows_v, bias_v, consts_v, out_v, sem, semi):
    cid = lax.axis_index("c")
    sid = lax.axis_index("s")
    wid = sid * NC + cid
    base = wid * BPW
    pltpu.sync_copy(consts_hbm, consts_v)

    icopies = [
        pltpu.async_copy(xf_hbm.at[pl.ds(f * B + base, BPW)],
                         widx_v.at[pl.ds(f * BPW, BPW)], semi)
        for f in range(F)
    ]
    for cp in icopies:
        cp.wait()

    def remap_body(i, _):
        v = widx_v[pl.ds(i * L, L)]
        r = ((v & -1024) | ((v & 127) << 3)
             | ((v >> 7) & 7))
        widx2_v[pl.ds(i * L, L)] = r
        return _

    lax.fori_loop(0, F * BPW // L, remap_body, None)

    iota = lax.broadcasted_iota(jnp.int32, (L,), 0)
    fi1 = iota * CH
    fi2 = fi1 + 16 * CH
    m0 = jnp.where(iota < F - L, 1.0, 0.0).astype(jnp.float32)
    w0row = consts_v[OFF_W0, :]
    ddrows = [consts_v[OFF_DD + f, :] for f in range(F)]
    derows = [consts_v[OFF_DE + c, :] for c in range(C)]

    def chunk_body(ch, _):
        copies = []
        for f in range(F):
            isl = widx_v.at[pl.ds(f * BPW, BPW)].at[pl.ds(ch * CH, CH)]
            isl2 = widx2_v.at[pl.ds(f * BPW, BPW)].at[pl.ds(ch * CH, CH)]
            copies.append(pltpu.async_copy(
                emb_hbm.at[isl2], rows_v.at[pl.ds(f * CH, CH)], sem))
            copies.append(pltpu.async_copy(
                bias_hbm.at[isl], bias_v.at[pl.ds(f * CH, CH)], sem))
        for cp in copies:
            cp.wait()

        def q_body(q, _):
            def k_body(k, resvec):
                b0 = q * L + k * 4
                accs = []
                for j in range(4):
                    row = rows_v[b0 + j, :]
                    accs.append(row * row * ddrows[0])
                Pss = [[rows_v[b0 + j, :] * consts_v[c, :] for c in range(C)]
                       for j in range(4)]
                for f in range(1, F):
                    rb = f * CH + b0
                    urows = [consts_v[f * C + c, :] for c in range(C)]
                    for j in range(4):
                        row = rows_v[rb + j, :]
                        accs[j] = accs[j] + row * row * ddrows[f]
                        Ps = Pss[j]
                        for c in range(C):
                            Ps[c] = Ps[c] + row * urows[c]
                for j in range(4):
                    for c in range(C):
                        accs[j] = accs[j] + Pss[j][c] * Pss[j][c] * derows[c]
                for j in range(4):
                    b = b0 + j
                    v1 = plsc.load_gather(bias_v, [b + fi1])
                    v2 = plsc.load_gather(bias_v, [jnp.minimum(b + fi2, ROWS - 1)])
                    tot = accs[j] + (v1 + v2 * m0) + w0row
                    r = jnp.sum(tot)
                    resvec = jnp.where(iota == k * 4 + j, r, resvec)
                return resvec

            resvec = lax.fori_loop(0, 4, k_body, jnp.zeros((L,), jnp.float32))
            out_v[pl.ds(ch * CH + q * L, L)] = resvec
            return _

        lax.fori_loop(0, CH // L, q_body, None)
        return _

    lax.fori_loop(0, NCHUNK, chunk_body, None)
    pltpu.sync_copy(out_v, out_hbm.at[pl.ds(base, BPW)])


def kernel(x, emb_table, bias_table, w0, diag_e, U):
    diag_d = -(diag_e[:, None] * U * U).sum(axis=0)
    ones = jnp.ones((1, L), jnp.float32)
    u_rows = U.T.reshape(-1, 1) * ones
    dd_rows = (0.5 * diag_d).reshape(-1, 1) * ones
    de_rows = (0.5 * diag_e).reshape(-1, 1) * ones
    w0_row = jnp.zeros((1, L), jnp.float32).at[0, 0].set(w0[0])
    consts = jnp.concatenate([u_rows, dd_rows, de_rows, w0_row], axis=0)

    table_rm = _transpose_table(emb_table.T)
    xf = x.astype(jnp.int32).T.reshape(-1)
    bias1d = bias_table.reshape(-1)

    mesh = plsc.VectorSubcoreMesh(core_axis_name="c", subcore_axis_name="s")
    fm = functools.partial(
        pl.kernel,
        mesh=mesh,
        compiler_params=pltpu.CompilerParams(
            needs_layout_passes=False, use_tc_tiling_on_sc=False),
        out_type=jax.ShapeDtypeStruct((B,), jnp.float32),
        scratch_types=[
            pltpu.VMEM((F * BPW,), jnp.int32),
            pltpu.VMEM((F * BPW,), jnp.int32),
            pltpu.VMEM((ROWS, D), jnp.float32),
            pltpu.VMEM((ROWS,), jnp.float32),
            pltpu.VMEM((NCONST, L), jnp.float32),
            pltpu.VMEM((BPW,), jnp.float32),
            pltpu.SemaphoreType.DMA,
            pltpu.SemaphoreType.DMA,
        ],
    )(_fm_body)
    return fm(xf, table_rm.reshape(VPAD, D), bias1d, consts)


# speedup vs baseline: 4.6526x; 1.1776x over previous
import functools

import jax
import jax.numpy as jnp
from jax import lax
from jax.experimental import pallas as pl
from jax.experimental.pallas import tpu as pltpu
from jax.experimental.pallas import tpu_sc as plsc

B = 16384
F = 26
D = 16
C = 8
V = 1000000
L = 16
NC = 2
NS = 16
NW = NC * NS
BPW = B // NW
CH = 128
NCHUNK = BPW // CH
ROWS = CH * F

OFF_DD = C * F
OFF_DE = OFF_DD + F
OFF_W0 = OFF_DE + C
NCONST = 243

GRP = 16
TCOLS = GRP * 8 * 128
TROWS = GRP * 128
TSTEPS = (V + TCOLS - 1) // TCOLS
VPAD = TSTEPS * TCOLS


def _tr_body(in_ref, bias_ref, out_ref, bias_out_ref):
    blk = in_ref[...]
    for c in range(GRP):
        tile = jnp.concatenate(
            [blk[:, (c * 8 + k) * 128:(c * 8 + k + 1) * 128]
             for k in range(8)], axis=0)
        out_ref[c * 128:(c + 1) * 128, :] = tile.T
    bias_out_ref[...] = bias_ref[0, :]


def _transpose_table(embT, biasT):
    return pl.pallas_call(
        _tr_body,
        grid=(TSTEPS,),
        in_specs=[pl.BlockSpec((D, TCOLS), lambda i: (0, i)),
                  pl.BlockSpec((1, TCOLS), lambda i: (0, i))],
        out_specs=[pl.BlockSpec((TROWS, 128), lambda i: (i, 0)),
                   pl.BlockSpec((TCOLS,), lambda i: (i,))],
        out_shape=[jax.ShapeDtypeStruct((VPAD * D // 128, 128), jnp.float32),
                   jax.ShapeDtypeStruct((VPAD,), jnp.float32)],
    )(embT, biasT)


def _fm_body(xf_hbm, emb_hbm, bias_hbm, consts_hbm, out_hbm,
             widx_v, widx2_v, rows_v, bias_v, consts_v, out_v, sem, semi):
    cid = lax.axis_index("c")
    sid = lax.axis_index("s")
    wid = sid * NC + cid
    base = wid * BPW
    pltpu.sync_copy(consts_hbm, consts_v)

    icopies = [
        pltpu.async_copy(xf_hbm.at[pl.ds(f * B + base, BPW)],
                         widx_v.at[pl.ds(f * BPW, BPW)], semi)
        for f in range(F)
    ]
    for cp in icopies:
        cp.wait()

    def remap_body(i, _):
        v = widx_v[pl.ds(i * L, L)]
        r = ((v & -1024) | ((v & 127) << 3)
             | ((v >> 7) & 7))
        widx2_v[pl.ds(i * L, L)] = r
        return _

    lax.fori_loop(0, F * BPW // L, remap_body, None)

    iota = lax.broadcasted_iota(jnp.int32, (L,), 0)
    fi1 = iota * CH
    fi2 = fi1 + 16 * CH
    m0 = jnp.where(iota < F - L, 1.0, 0.0).astype(jnp.float32)
    w0row = consts_v[OFF_W0, :]
    ddrows = [consts_v[OFF_DD + f, :] for f in range(F)]
    derows = [consts_v[OFF_DE + c, :] for c in range(C)]

    def chunk_body(ch, _):
        copies = []
        for f in range(F):
            isl = widx_v.at[pl.ds(f * BPW, BPW)].at[pl.ds(ch * CH, CH)]
            isl2 = widx2_v.at[pl.ds(f * BPW, BPW)].at[pl.ds(ch * CH, CH)]
            copies.append(pltpu.async_copy(
                emb_hbm.at[isl2], rows_v.at[pl.ds(f * CH, CH)], sem))
            copies.append(pltpu.async_copy(
                bias_hbm.at[isl], bias_v.at[pl.ds(f * CH, CH)], sem))
        for cp in copies:
            cp.wait()

        def q_body(q, _):
            def k_body(k, resvec):
                b0 = q * L + k * 4
                accs = []
                for j in range(4):
                    row = rows_v[b0 + j, :]
                    accs.append(row * row * ddrows[0])
                Pss = [[rows_v[b0 + j, :] * consts_v[c, :] for c in range(C)]
                       for j in range(4)]
                for f in range(1, F):
                    rb = f * CH + b0
                    urows = [consts_v[f * C + c, :] for c in range(C)]
                    for j in range(4):
                        row = rows_v[rb + j, :]
                        accs[j] = accs[j] + row * row * ddrows[f]
                        Ps = Pss[j]
                        for c in range(C):
                            Ps[c] = Ps[c] + row * urows[c]
                for j in range(4):
                    for c in range(C):
                        accs[j] = accs[j] + Pss[j][c] * Pss[j][c] * derows[c]
                for j in range(4):
                    b = b0 + j
                    v1 = plsc.load_gather(bias_v, [b + fi1])
                    v2 = plsc.load_gather(bias_v, [jnp.minimum(b + fi2, ROWS - 1)])
                    tot = accs[j] + (v1 + v2 * m0) + w0row
                    r = jnp.sum(tot)
                    resvec = jnp.where(iota == k * 4 + j, r, resvec)
                return resvec

            resvec = lax.fori_loop(0, 4, k_body, jnp.zeros((L,), jnp.float32))
            out_v[pl.ds(ch * CH + q * L, L)] = resvec
            return _

        lax.fori_loop(0, CH // L, q_body, None)
        return _

    lax.fori_loop(0, NCHUNK, chunk_body, None)
    pltpu.sync_copy(out_v, out_hbm.at[pl.ds(base, BPW)])


def kernel(x, emb_table, bias_table, w0, diag_e, U):
    diag_d = -(diag_e[:, None] * U * U).sum(axis=0)
    ones = jnp.ones((1, L), jnp.float32)
    u_rows = U.T.reshape(-1, 1) * ones
    dd_rows = (0.5 * diag_d).reshape(-1, 1) * ones
    de_rows = (0.5 * diag_e).reshape(-1, 1) * ones
    w0_row = jnp.zeros((1, L), jnp.float32).at[0, 0].set(w0[0])
    consts = jnp.concatenate([u_rows, dd_rows, de_rows, w0_row], axis=0)

    table_rm, bias1d = _transpose_table(emb_table.T, bias_table.T)
    xf = x.astype(jnp.int32).T.reshape(-1)

    mesh = plsc.VectorSubcoreMesh(core_axis_name="c", subcore_axis_name="s")
    fm = functools.partial(
        pl.kernel,
        mesh=mesh,
        compiler_params=pltpu.CompilerParams(
            needs_layout_passes=False, use_tc_tiling_on_sc=False),
        out_type=jax.ShapeDtypeStruct((B,), jnp.float32),
        scratch_types=[
            pltpu.VMEM((F * BPW,), jnp.int32),
            pltpu.VMEM((F * BPW,), jnp.int32),
            pltpu.VMEM((ROWS, D), jnp.float32),
            pltpu.VMEM((ROWS,), jnp.float32),
            pltpu.VMEM((NCONST, L), jnp.float32),
            pltpu.VMEM((BPW,), jnp.float32),
            pltpu.SemaphoreType.DMA,
            pltpu.SemaphoreType.DMA,
        ],
    )(_fm_body)
    return fm(xf, table_rm.reshape(VPAD, D), bias1d, consts)
